# Initial kernel scaffold; baseline (speedup 1.0000x reference)
import jax
import jax.numpy as jnp
from jax.experimental import pallas as pl


def kernel(x, flat):
    raise NotImplementedError("write your pallas kernel here")


# speedup vs baseline: 1.2836x; 1.2836x over previous
import functools

import jax
import jax.numpy as jnp
import numpy as np
from jax import lax
from jax.experimental import pallas as pl
from jax.experimental.pallas import tpu as pltpu
from jax.experimental.pallas import tpu_sc as plsc

_TOPK = 12
_LANES = 16


def _scores_body(x_ref, wk_ref, wq_ref, a_ref, *, scale):
    x = x_ref[0]
    xk = jnp.dot(x, wk_ref[...], preferred_element_type=jnp.float32)
    xq = jnp.dot(x, wq_ref[...], preferred_element_type=jnp.float32)
    h = lax.dot_general(xk, xq, (((1,), (1,)), ((), ())),
                        preferred_element_type=jnp.float32)
    s = scale * h
    m = jnp.max(s, axis=-1, keepdims=True)
    e = jnp.exp(s - m)
    p = e / jnp.sum(e, axis=-1, keepdims=True)
    a_ref[0, 0, :] = jnp.sum(p, axis=0)


def _topk_body(rows_per_worker, n, a_hbm, out_hbm, row_v, out_v):
    wid = lax.axis_index("s") * 2 + lax.axis_index("c")
    nchunk = n // _LANES
    iota = lax.iota(jnp.int32, _LANES)
    neg = jnp.float32(-jnp.inf)

    def do_row(r, _):
        row = wid * rows_per_worker + r
        pltpu.sync_copy(a_hbm.at[row], row_v)

        keys = jnp.full((_LANES,), neg, jnp.float32)
        vals = jnp.zeros((_LANES,), jnp.int32)
        for c in range(nchunk):
            v = row_v[pl.ds(c * _LANES, _LANES)]
            gidx = c * _LANES + iota
            vs, vi = plsc.sort_key_val(v, gidx, descending=False)
            take = keys >= vs
            mk = jnp.where(take, keys, vs)
            mv = jnp.where(take, vals, vi)
            keys, vals = plsc.sort_key_val(mk, mv, descending=True)
        out_v[...] = vals
        pltpu.sync_copy(out_v, out_hbm.at[row])
        return 0

    lax.fori_loop(0, rows_per_worker, do_row, 0)


def kernel(x, flat):
    N, T, n, d_in = x.shape
    d = flat.shape[0] // (2 * d_in)
    B = N * T
    wk = flat[: d_in * d].reshape(d_in, d)
    wq = flat[d_in * d:].reshape(d_in, d)
    xf = x.reshape(B, n, d_in)
    scale = np.float32(1.0 / np.sqrt(np.float32(d_in)))

    a = pl.pallas_call(
        functools.partial(_scores_body, scale=scale),
        grid=(B,),
        in_specs=[
            pl.BlockSpec((1, n, d_in), lambda b: (b, 0, 0)),
            pl.BlockSpec((d_in, d), lambda b: (0, 0)),
            pl.BlockSpec((d_in, d), lambda b: (0, 0)),
        ],
        out_specs=pl.BlockSpec((1, 1, n), lambda b: (b, 0, 0)),
        out_shape=jax.ShapeDtypeStruct((B, 1, n), jnp.float32),
    )(xf, wk, wq)
    a = a.reshape(B, n)

    info = plsc.get_sparse_core_info()
    nworkers = info.num_cores * info.num_subcores
    rows_per_worker = B // nworkers
    mesh = plsc.VectorSubcoreMesh(core_axis_name="c", subcore_axis_name="s")

    topk = pl.kernel(
        functools.partial(_topk_body, rows_per_worker, n),
        out_type=jax.ShapeDtypeStruct((B, _LANES), jnp.int32),
        mesh=mesh,
        compiler_params=pltpu.CompilerParams(needs_layout_passes=False),
        scratch_types=[
            pltpu.VMEM((n,), jnp.float32),
            pltpu.VMEM((_LANES,), jnp.int32),
        ],
    )(a)

    return topk[:, :_TOPK].reshape(N, T, _TOPK, 1)


# speedup vs baseline: 2.1226x; 1.6537x over previous
import functools

import jax
import jax.numpy as jnp
import numpy as np
from jax import lax
from jax.experimental import pallas as pl
from jax.experimental.pallas import tpu as pltpu
from jax.experimental.pallas import tpu_sc as plsc

_TOPK = 12
_LANES = 16


def _scores_body(x_ref, wk_ref, wq_ref, a_ref, *, scale, group):
    for g in range(group):
        x = x_ref[g]
        xk = jnp.dot(x, wk_ref[...], preferred_element_type=jnp.float32)
        xq = jnp.dot(x, wq_ref[...], preferred_element_type=jnp.float32)
        h = lax.dot_general(xk, xq, (((1,), (1,)), ((), ())),
                            preferred_element_type=jnp.float32)
        s = scale * h
        m = jnp.max(s, axis=-1, keepdims=True)
        e = jnp.exp(s - m)
        p = e / jnp.sum(e, axis=-1, keepdims=True)
        a_ref[g, 0, :] = jnp.sum(p, axis=0)


def _topk_body(rows_per_worker, n, a_hbm, out_hbm, row_v, out_v):
    wid = lax.axis_index("s") * 2 + lax.axis_index("c")
    nchunk = n // _LANES
    iota = lax.iota(jnp.int32, _LANES)
    neg = jnp.float32(-jnp.inf)

    def do_row(r, _):
        row = wid * rows_per_worker + r
        pltpu.sync_copy(a_hbm.at[row], row_v)

        keys = jnp.full((_LANES,), neg, jnp.float32)
        vals = jnp.zeros((_LANES,), jnp.int32)
        for c in range(nchunk):
            v = row_v[pl.ds(c * _LANES, _LANES)]
            gidx = c * _LANES + iota
            vs, vi = plsc.sort_key_val(v, gidx, descending=False)
            take = keys >= vs
            mk = jnp.where(take, keys, vs)
            mv = jnp.where(take, vals, vi)
            keys, vals = plsc.sort_key_val(mk, mv, descending=True)
        out_v[...] = vals
        pltpu.sync_copy(out_v, out_hbm.at[row])
        return 0

    lax.fori_loop(0, rows_per_worker, do_row, 0)


def kernel(x, flat):
    N, T, n, d_in = x.shape
    d = flat.shape[0] // (2 * d_in)
    B = N * T
    wk = flat[: d_in * d].reshape(d_in, d)
    wq = flat[d_in * d:].reshape(d_in, d)
    xf = x.reshape(B, n, d_in)
    scale = np.float32(1.0 / np.sqrt(np.float32(d_in)))

    group = 8
    a = pl.pallas_call(
        functools.partial(_scores_body, scale=scale, group=group),
        grid=(B // group,),
        in_specs=[
            pl.BlockSpec((group, n, d_in), lambda b: (b, 0, 0)),
            pl.BlockSpec((d_in, d), lambda b: (0, 0)),
            pl.BlockSpec((d_in, d), lambda b: (0, 0)),
        ],
        out_specs=pl.BlockSpec((group, 1, n), lambda b: (b, 0, 0)),
        out_shape=jax.ShapeDtypeStruct((B, 1, n), jnp.float32),
    )(xf, wk, wq)
    a = a.reshape(B, n)

    info = plsc.get_sparse_core_info()
    nworkers = info.num_cores * info.num_subcores
    rows_per_worker = B // nworkers
    mesh = plsc.VectorSubcoreMesh(core_axis_name="c", subcore_axis_name="s")

    topk = pl.kernel(
        functools.partial(_topk_body, rows_per_worker, n),
        out_type=jax.ShapeDtypeStruct((B, _LANES), jnp.int32),
        mesh=mesh,
        compiler_params=pltpu.CompilerParams(needs_layout_passes=False),
        scratch_types=[
            pltpu.VMEM((n,), jnp.float32),
            pltpu.VMEM((_LANES,), jnp.int32),
        ],
    )(a)

    return topk[:, :_TOPK].reshape(N, T, _TOPK, 1)


# speedup vs baseline: 2.4920x; 1.1741x over previous
import functools

import jax
import jax.numpy as jnp
import numpy as np
from jax import lax
from jax.experimental import pallas as pl
from jax.experimental.pallas import tpu as pltpu
from jax.experimental.pallas import tpu_sc as plsc

_TOPK = 12
_LANES = 16


def _scores_body(x_ref, wkq_ref, a_ref, *, group, n, d):
    xall = x_ref[...].reshape(group * n, x_ref.shape[2])
    kqt = lax.dot_general(wkq_ref[...], xall, (((0,), (1,)), ((), ())),
                          preferred_element_type=jnp.float32)
    for g in range(group):
        xkt = kqt[:d, g * n:(g + 1) * n]
        xqt = kqt[d:, g * n:(g + 1) * n]
        s = lax.dot_general(xkt, xqt, (((0,), (0,)), ((), ())),
                            preferred_element_type=jnp.float32)
        m = jnp.max(s, axis=-1, keepdims=True)
        e = jnp.exp(s - m)
        p = e / jnp.sum(e, axis=-1, keepdims=True)
        a_ref[g, 0, :] = jnp.sum(p, axis=0)


def _topk_body(rows_per_worker, n, a_hbm, out_hbm, row_v, out_v):
    wid = lax.axis_index("s") * 2 + lax.axis_index("c")
    nchunk = n // _LANES
    iota = lax.iota(jnp.int32, _LANES)
    neg = jnp.float32(-jnp.inf)

    def do_row(r, _):
        row = wid * rows_per_worker + r
        pltpu.sync_copy(a_hbm.at[row], row_v)

        keys = jnp.full((_LANES,), neg, jnp.float32)
        vals = jnp.zeros((_LANES,), jnp.int32)
        for c in range(nchunk):
            v = row_v[pl.ds(c * _LANES, _LANES)]
            gidx = c * _LANES + iota
            vs, vi = plsc.sort_key_val(v, gidx, descending=False)
            take = keys >= vs
            mk = jnp.where(take, keys, vs)
            mv = jnp.where(take, vals, vi)
            keys, vals = plsc.sort_key_val(mk, mv, descending=True)
        out_v[...] = vals
        pltpu.sync_copy(out_v, out_hbm.at[row])
        return 0

    lax.fori_loop(0, rows_per_worker, do_row, 0)


def kernel(x, flat):
    N, T, n, d_in = x.shape
    d = flat.shape[0] // (2 * d_in)
    B = N * T
    wk = flat[: d_in * d].reshape(d_in, d)
    wq = flat[d_in * d:].reshape(d_in, d)
    xf = x.reshape(B, n, d_in)
    scale = np.float32(1.0 / np.sqrt(np.float32(d_in)))

    group = 8
    wkq = jnp.concatenate([wk, wq * scale], axis=1)
    a = pl.pallas_call(
        functools.partial(_scores_body, group=group, n=n, d=d),
        grid=(B // group,),
        in_specs=[
            pl.BlockSpec((group, n, d_in), lambda b: (b, 0, 0)),
            pl.BlockSpec((d_in, 2 * d), lambda b: (0, 0)),
        ],
        out_specs=pl.BlockSpec((group, 1, n), lambda b: (b, 0, 0)),
        out_shape=jax.ShapeDtypeStruct((B, 1, n), jnp.float32),
    )(xf, wkq)
    a = a.reshape(B, n)

    info = plsc.get_sparse_core_info()
    nworkers = info.num_cores * info.num_subcores
    rows_per_worker = B // nworkers
    mesh = plsc.VectorSubcoreMesh(core_axis_name="c", subcore_axis_name="s")

    topk = pl.kernel(
        functools.partial(_topk_body, rows_per_worker, n),
        out_type=jax.ShapeDtypeStruct((B, _LANES), jnp.int32),
        mesh=mesh,
        compiler_params=pltpu.CompilerParams(needs_layout_passes=False),
        scratch_types=[
            pltpu.VMEM((n,), jnp.float32),
            pltpu.VMEM((_LANES,), jnp.int32),
        ],
    )(a)

    return topk[:, :_TOPK].reshape(N, T, _TOPK, 1)
